# baseline (speedup 1.0000x reference)
import jax
import jax.numpy as jnp
from jax import lax
from jax.experimental import pallas as pl
from jax.experimental.pallas import tpu as pltpu
from jax.experimental.pallas import tpu_sc as plsc

EMB = 1024
PAD = 1
L = 16
NC, NS = 2, 16
NW = NC * NS
BSZ, SEQ = 4, 4096
ROWS = BSZ * SEQ
RPW = ROWS // NW
WPB = NW // BSZ
CPW = SEQ // WPB
CH = 32
NG = RPW // CH
NB = 3


def _body(tok_hbm, w_hbm, out_hbm, tok_v, idx_v, buf0, buf1, buf2,
          gs0, gs1, gs2, ws0, ws1, ws2):
    wid = lax.axis_index("s") * NC + lax.axis_index("c")
    b = wid // WPB
    c = wid % WPB
    pltpu.sync_copy(tok_hbm.at[b], tok_v)

    def pre(i, acc):
        g0 = tok_v[pl.ds(i * 4 * L, L)]
        g1 = tok_v[pl.ds(i * 4 * L + L, L)]
        g2 = tok_v[pl.ds(i * 4 * L + 2 * L, L)]
        g3 = tok_v[pl.ds(i * 4 * L + 3 * L, L)]
        acc = acc + jnp.where(g0 == PAD, 0, 1) + jnp.where(g1 == PAD, 0, 1)
        return acc + jnp.where(g2 == PAD, 0, 1) + jnp.where(g3 == PAD, 0, 1)

    acc = lax.fori_loop(0, c * (CPW // (4 * L)), pre,
                        jnp.zeros((L,), jnp.int32))
    carry0 = jnp.sum(acc)

    def pos_chunk(g, carry):
        tb = c * CPW + g * CH
        grp0 = tok_v[pl.ds(tb, L)]
        m0 = jnp.where(grp0 == PAD, 0, 1)
        cs0 = plsc.cumsum(m0)
        idx_v[pl.ds(g * CH, L)] = (carry + cs0) * m0 + PAD
        c1 = carry + jnp.sum(m0)
        grp1 = tok_v[pl.ds(tb + L, L)]
        m1 = jnp.where(grp1 == PAD, 0, 1)
        cs1 = plsc.cumsum(m1)
        idx_v[pl.ds(g * CH + L, L)] = (c1 + cs1) * m1 + PAD
        return c1 + jnp.sum(m1)

    base = wid * RPW
    bufs = (buf0, buf1, buf2)
    gsems = (gs0, gs1, gs2)
    wsems = (ws0, ws1, ws2)

    def start(g):
        return pltpu.async_copy(w_hbm.at[idx_v.at[pl.ds(g * CH, CH)]],
                                bufs[g % NB], gsems[g % NB])

    carry = pos_chunk(0, carry0)
    carry = pos_chunk(1, carry)
    gh = [None] * NG
    wh = [None] * NG
    gh[0] = start(0)
    gh[1] = start(1)
    for g in range(NG):
        p = g % NB
        if g + 2 < NG:
            carry = pos_chunk(g + 2, carry)
            if g - 1 >= 0:
                wh[g - 1].wait()
            gh[g + 2] = start(g + 2)
        gh[g].wait()
        wh[g] = pltpu.async_copy(bufs[p],
                                 out_hbm.at[pl.ds(base + g * CH, CH)],
                                 wsems[p])
    for k in range(NB):
        wh[NG - NB + k].wait()


@jax.jit
def _sc_embed(tokens, weights):
    mesh = plsc.VectorSubcoreMesh(core_axis_name="c", subcore_axis_name="s",
                                  num_cores=NC, num_subcores=NS)
    return pl.kernel(
        _body,
        out_type=jax.ShapeDtypeStruct((ROWS, EMB), jnp.float32),
        mesh=mesh,
        compiler_params=pltpu.CompilerParams(needs_layout_passes=False,
                                             skip_device_barrier=True),
        scratch_types=[
            pltpu.VMEM((SEQ,), jnp.int32),
            pltpu.VMEM((RPW,), jnp.int32),
            pltpu.VMEM((CH, EMB), jnp.float32),
            pltpu.VMEM((CH, EMB), jnp.float32),
            pltpu.VMEM((CH, EMB), jnp.float32),
            pltpu.SemaphoreType.DMA,
            pltpu.SemaphoreType.DMA,
            pltpu.SemaphoreType.DMA,
            pltpu.SemaphoreType.DMA,
            pltpu.SemaphoreType.DMA,
            pltpu.SemaphoreType.DMA,
        ],
    )(tokens, weights)


def kernel(input, weights):
    bsz, seq_len = input.shape
    out = _sc_embed(input, weights)
    return lax.stop_gradient(out.reshape(bsz, seq_len, -1))


# speedup vs baseline: 1.0124x; 1.0124x over previous
import jax
import jax.numpy as jnp
from jax import lax
from jax.experimental import pallas as pl
from jax.experimental.pallas import tpu as pltpu
from jax.experimental.pallas import tpu_sc as plsc

EMB = 1024
PAD = 1
L = 16
NC, NS = 2, 16
NW = NC * NS
BSZ, SEQ = 4, 4096
ROWS = BSZ * SEQ
RPW = ROWS // NW
WPB = NW // BSZ
CPW = SEQ // WPB
CH = 16
NG = RPW // CH
NB = 6
GPC = CH // L


def _body(tok_hbm, w_hbm, out_hbm, tok_v, idx_v,
          b0, b1, b2, b3, b4, b5,
          g0, g1, g2, g3, g4, g5,
          w0, w1, w2, w3, w4, w5):
    wid = lax.axis_index("s") * NC + lax.axis_index("c")
    b = wid // WPB
    c = wid % WPB
    pltpu.sync_copy(tok_hbm.at[b], tok_v)

    def pre(i, acc):
        q0 = tok_v[pl.ds(i * 4 * L, L)]
        q1 = tok_v[pl.ds(i * 4 * L + L, L)]
        q2 = tok_v[pl.ds(i * 4 * L + 2 * L, L)]
        q3 = tok_v[pl.ds(i * 4 * L + 3 * L, L)]
        acc = acc + jnp.where(q0 == PAD, 0, 1) + jnp.where(q1 == PAD, 0, 1)
        return acc + jnp.where(q2 == PAD, 0, 1) + jnp.where(q3 == PAD, 0, 1)

    acc = lax.fori_loop(0, c * (CPW // (4 * L)), pre,
                        jnp.zeros((L,), jnp.int32))
    carry0 = jnp.sum(acc)

    def pos_chunk(g, carry):
        for q in range(GPC):
            grp = tok_v[pl.ds(c * CPW + g * CH + q * L, L)]
            m = jnp.where(grp == PAD, 0, 1)
            cs = plsc.cumsum(m)
            idx_v[pl.ds(g * CH + q * L, L)] = (carry + cs) * m + PAD
            carry = carry + jnp.sum(m)
        return carry

    base = wid * RPW
    bufs = (b0, b1, b2, b3, b4, b5)[:NB]
    gsems = (g0, g1, g2, g3, g4, g5)[:NB]
    wsems = (w0, w1, w2, w3, w4, w5)[:NB]
    P = NB - 1

    def start(g):
        return pltpu.async_copy(w_hbm.at[idx_v.at[pl.ds(g * CH, CH)]],
                                bufs[g % NB], gsems[g % NB])

    carry = carry0
    gh = [None] * NG
    wh = [None] * NG
    for g in range(P):
        carry = pos_chunk(g, carry)
        gh[g] = start(g)
    for g in range(NG):
        p = g % NB
        if g + P < NG:
            carry = pos_chunk(g + P, carry)
            if g - 1 >= 0:
                wh[g - 1].wait()
            gh[g + P] = start(g + P)
        gh[g].wait()
        wh[g] = pltpu.async_copy(bufs[p],
                                 out_hbm.at[pl.ds(base + g * CH, CH)],
                                 wsems[p])
    for k in range(NB):
        wh[NG - NB + k].wait()


@jax.jit
def _sc_embed(tokens, weights):
    mesh = plsc.VectorSubcoreMesh(core_axis_name="c", subcore_axis_name="s",
                                  num_cores=NC, num_subcores=NS)
    return pl.kernel(
        _body,
        out_type=jax.ShapeDtypeStruct((ROWS, EMB), jnp.float32),
        mesh=mesh,
        compiler_params=pltpu.CompilerParams(needs_layout_passes=False,
                                             skip_device_barrier=True),
        scratch_types=(
            [pltpu.VMEM((SEQ,), jnp.int32),
             pltpu.VMEM((RPW,), jnp.int32)]
            + [pltpu.VMEM((CH, EMB), jnp.float32)] * NB
            + [pltpu.SemaphoreType.DMA] * (2 * NB)
        ),
    )(tokens, weights)


def kernel(input, weights):
    bsz, seq_len = input.shape
    out = _sc_embed(input, weights)
    return lax.stop_gradient(out.reshape(bsz, seq_len, -1))
